# Initial kernel scaffold; baseline (speedup 1.0000x reference)
import jax
import jax.numpy as jnp
from jax.experimental import pallas as pl


def kernel(particles, noise_u, resample_offsets, noise_x, K, log_sig, Q, R, A, B, iteration):
    raise NotImplementedError("write your pallas kernel here")


# speedup vs baseline: 2.7935x; 2.7935x over previous
import functools

import jax
import jax.numpy as jnp
from jax import lax
from jax.scipy.special import logsumexp
from jax.experimental import pallas as pl
from jax.experimental.pallas import tpu as pltpu
from jax.experimental.pallas import tpu_sc as plsc

_NUM_P = 65536
_U = 8
_DX = 128
_DU = 32
_ALPHA = 1.0
_EXP_FACTOR = 2.0
_NU = _NUM_P * _U

_NC = 2
_NS = 16
_NW = _NC * _NS
_ROWS_W = _NUM_P // _NW
_CHUNK = 128
_NCHUNK = _ROWS_W // _CHUNK


_sc_mesh = plsc.VectorSubcoreMesh(core_axis_name="c", subcore_axis_name="s")


def _sc_gather_body(particles_hbm, g_hbm,
                    x_out,
                    idx_g, xbuf, sem_x):
    wid = lax.axis_index("s") * _NC + lax.axis_index("c")
    base0 = wid * _ROWS_W
    row0 = wid * _NCHUNK
    pltpu.sync_copy(g_hbm.at[pl.ds(row0, _NCHUNK)], idx_g)

    def body(ci, carry):
        base = base0 + ci * _CHUNK
        pltpu.async_copy(particles_hbm.at[idx_g.at[ci]], xbuf, sem_x).wait()
        pltpu.sync_copy(xbuf, x_out.at[pl.ds(base, _CHUNK)])
        return carry

    lax.fori_loop(0, _NCHUNK, body, 0)


def _make_sc_gather(interpret=False):
    return pl.kernel(
        _sc_gather_body,
        out_type=(
            jax.ShapeDtypeStruct((_NUM_P, _DX), jnp.float32),
        ),
        mesh=_sc_mesh,
        interpret=interpret,
        scratch_types=[
            pltpu.VMEM((_NCHUNK, _CHUNK), jnp.int32),
            pltpu.VMEM((_CHUNK, _DX), jnp.float32),
            pltpu.SemaphoreType.DMA,
        ],
    )


_sc_gather = _make_sc_gather()


_BR = 1024


def _dyn_body(x_ref, u_ref, nx_ref, a_ref, b_ref, np_ref, cat_ref):
    x = x_ref[...]
    u = u_ref[...]
    np_ref[...] = (
        jnp.dot(x, a_ref[...], preferred_element_type=jnp.float32,
                precision=lax.Precision.HIGHEST)
        + jnp.dot(u, b_ref[...], preferred_element_type=jnp.float32,
                  precision=lax.Precision.HIGHEST)
        + nx_ref[...]
    )
    cat_ref[...] = jnp.concatenate([x, u], axis=1)


def _dynamics(x_sel, u_sel, noise_x, A, B):
    return pl.pallas_call(
        _dyn_body,
        grid=(_NUM_P // _BR,),
        in_specs=[
            pl.BlockSpec((_BR, _DX), lambda i: (i, 0)),
            pl.BlockSpec((_BR, _DU), lambda i: (i, 0)),
            pl.BlockSpec((_BR, _DX), lambda i: (i, 0)),
            pl.BlockSpec((_DX, _DX), lambda i: (0, 0)),
            pl.BlockSpec((_DU, _DX), lambda i: (0, 0)),
        ],
        out_specs=[
            pl.BlockSpec((_BR, _DX), lambda i: (i, 0)),
            pl.BlockSpec((_BR, _DX + _DU), lambda i: (i, 0)),
        ],
        out_shape=[
            jax.ShapeDtypeStruct((_NUM_P, _DX), jnp.float32),
            jax.ShapeDtypeStruct((_NUM_P, _DX + _DU), jnp.float32),
        ],
    )(x_sel, u_sel, noise_x, A, B)


def kernel(particles, noise_u, resample_offsets, noise_x, K, log_sig, Q, R, A, B, iteration):
    mu = particles @ K
    mu_rep = jnp.repeat(mu, _U, axis=0)
    x_rep = jnp.repeat(particles, _U, axis=0)
    sig = jnp.exp(log_sig)
    new_u = mu_rep + sig * noise_u
    cost = 0.5 * jnp.sum((x_rep @ Q) * x_rep, axis=1) + 0.5 * jnp.sum((new_u @ R) * new_u, axis=1)
    u_corr = (-_EXP_FACTOR ** 2 + 1) / (2.0 * _EXP_FACTOR ** 2) * jnp.sum(((new_u - mu_rep) / sig) ** 2, axis=1)
    log_weights = -_ALPHA * cost + jnp.log(_EXP_FACTOR) + u_corr
    log_norm = log_weights - logsumexp(log_weights)
    cdf = jnp.cumsum(jnp.exp(log_norm))
    positions = (jnp.arange(_NUM_P, dtype=jnp.float32) + resample_offsets) / _NUM_P
    samples = jnp.clip(jnp.searchsorted(cdf, positions), 0, _NU - 1)
    samples_div = samples // _U

    (x_sel,) = _sc_gather(
        particles,
        samples_div.astype(jnp.int32).reshape(_NUM_P // _CHUNK, _CHUNK),
    )
    u_sel = new_u[samples]
    lw_sel = log_weights[samples].reshape(_NUM_P, 1)

    new_particles, particles_cat = _dynamics(x_sel, u_sel, noise_x, A, B)

    return (new_particles, particles_cat, lw_sel.reshape(_NUM_P), samples_div)
